# baseline (speedup 1.0000x reference)
import numpy as np

import jax
import jax.numpy as jnp
from jax.experimental import pallas as pl
from jax.experimental.pallas import tpu as pltpu

B = 2
N = 256
T = 24
F = 8
H = 128
FE = 8
POS = 32
O = 32
P = 8
TAILK = 12
BN = B * N

_EW_CHUNK = 8192
_NE = B * N * N

_SHARD = [4320, 4320] + [4080] * 13
_BOUND = np.cumsum(_SHARD)
_FIXES = []
for _b in range(B):
    for _u in _BOUND:
        _p = int(_u % 256)
        if _p != 0:
            _FIXES.append((_b * N + int(_u // 256), _p))
_NFIX = len(_FIXES)
_FIXB = _NFIX // 2
_FPAD = 16
_PFIX = np.full((1, 2 * _FPAD), 256, np.int32)
for _k, (_r, _p) in enumerate(_FIXES):
    _b = _k // _FIXB
    _PFIX[0, _b * _FPAD + (_k % _FIXB)] = _p


def _ew_kernel(ea_ref, md_ref, mu_ref, w1_ref, b1_ref, w2_ref, b2_ref, o_ref):
    h = jnp.dot(ea_ref[...], w1_ref[...], preferred_element_type=jnp.float32)
    h = jnp.maximum(h + b1_ref[...], 0.0)
    s = jnp.dot(h, w2_ref[...], preferred_element_type=jnp.float32) + b2_ref[...]
    ew = jax.nn.sigmoid(s)
    m = jnp.minimum(md_ref[...] + mu_ref[...], 1.0)
    o_ref[...] = m * ew


def _gru_kernel(at_ref, abt_ref, pfix_ref, xtp_ref, na_ref, outlet_ref,
                w_dyn_ref, b_dyn_ref, w_film_ref, b_film_ref,
                wzr0_ref, bzr0_ref, wc0_ref, bc0_ref,
                wzr1_ref, bzr1_ref, wc1_ref, bc1_ref,
                w_out_ref, b_out_ref,
                out_ref,
                s0_ref, s1_ref, hs_ref, xt_ref, ax_ref, ah_ref, acc_ref):
    f32 = jnp.float32

    film = jnp.dot(na_ref[...], w_film_ref[...], preferred_element_type=f32)
    film = film + b_film_ref[...]
    scale = 1.0 + film[:, :H]
    beta = film[:, H:]
    w_dyn = w_dyn_ref[...]
    b_dyn = b_dyn_ref[...]
    for t in range(T):
        xt = xtp_ref[t * F:(t + 1) * F, :]
        ht = jax.lax.dot_general(xt, w_dyn, (((0,), (0,)), ((), ())),
                                 preferred_element_type=f32)
        s0_ref[t] = jnp.maximum((ht + b_dyn) * scale + beta, 0.0)

    pfix = pfix_ref[...]

    def gprop(src_ref, dst_ref):
        HH = H // 2
        for b in range(B):
            pfb = pfix[:, b * _FPAD:(b + 1) * _FPAD]
            for hh in range(2):

                def gbody(g, carry, b=b, pfb=pfb, hh=hh):
                    accT, floT, fhiT = carry
                    blk = at_ref[b, pl.ds(g * 8, 8), :]
                    fblk = abt_ref[pl.ds(g * 8, 8),
                                   b * _FPAD:(b + 1) * _FPAD]
                    hb = src_ref[pl.ds(b * N + g * 8, 8), :]
                    hT = jnp.swapaxes(hb[:, hh * HH:(hh + 1) * HH],
                                      0, 1)
                    sel = (g * 8 < pfb).astype(f32)
                    selc = 1.0 - sel
                    for s in range(8):
                        row = blk[s:s + 1, :]
                        hc = hT[:, s:s + 1]
                        accT = accT + hc * row
                        mfx = hc * fblk[s:s + 1, :]
                        floT = floT + mfx * sel
                        fhiT = fhiT + mfx * selc
                    return accT, floT, fhiT

                zA = jnp.zeros((HH, N), f32)
                zF = jnp.zeros((HH, _FPAD), f32)
                accT, floT, fhiT = jax.lax.fori_loop(0, N // 8, gbody,
                                                     (zA, zF, zF))
                dst_ref[b * N:(b + 1) * N, hh * HH:(hh + 1) * HH] = (
                    jnp.swapaxes(accT, 0, 1))
                fix = jnp.swapaxes(floT + fhiT, 0, 1)
                for k in range(_FIXB):
                    r = _FIXES[b * _FIXB + k][0]
                    dst_ref[r:r + 1, hh * HH:(hh + 1) * HH] = fix[k:k + 1, :]

    acc_ref[...] = jnp.zeros((BN, H), f32)
    for layer in range(2):
        wzr = (wzr0_ref if layer == 0 else wzr1_ref)[...]
        bzr = (bzr0_ref if layer == 0 else bzr1_ref)[...]
        wc = (wc0_ref if layer == 0 else wc1_ref)[...]
        bc = (bc0_ref if layer == 0 else bc1_ref)[...]
        s_in = s0_ref if layer == 0 else s1_ref
        hs_ref[...] = jnp.zeros((BN, H), f32)

        def step(t, _, s_in=s_in, wzr=wzr, bzr=bzr, wc=wc, bc=bc, layer=layer):
            xt_ref[...] = s_in[t]
            gprop(xt_ref, ax_ref)
            gprop(hs_ref, ah_ref)
            xt = xt_ref[...]
            h = hs_ref[...]
            agg_x = ax_ref[...]
            agg_h = ah_ref[...]
            m = jnp.concatenate([xt, agg_x, h, agg_h], axis=1)
            zr = jax.nn.sigmoid(jnp.dot(m, wzr, preferred_element_type=f32)
                                + bzr)
            z = zr[:, :H]
            r = zr[:, H:]
            mc = jnp.concatenate([xt, agg_x, r * h, r * agg_h], axis=1)
            c = jnp.tanh(jnp.dot(mc, wc, preferred_element_type=f32) + bc)
            hnew = z * h + (1.0 - z) * c
            hs_ref[...] = hnew
            if layer == 0:
                s1_ref[t] = hnew
            else:
                acc_ref[...] = acc_ref[...] + jnp.where(t >= T - TAILK,
                                                        hnew, 0.0)
            return 0

        jax.lax.fori_loop(0, T, step, 0)

    tail = acc_ref[...] * (1.0 / TAILK)
    npred = jnp.dot(tail, w_out_ref[...], preferred_element_type=f32)
    npred = npred + b_out_ref[...]
    iota = jax.lax.broadcasted_iota(jnp.int32, (N, O), 0)
    for b in range(B):
        idx = outlet_ref[b:b + 1, :]
        onehot = (iota == idx).astype(f32)
        nb = npred[b * N:(b + 1) * N, :]
        yb = jax.lax.dot_general(nb, onehot, (((0,), (0,)), ((), ())),
                                 precision=jax.lax.Precision.HIGHEST,
                                 preferred_element_type=f32)
        out_ref[b] = yb


@jax.jit
def kernel(x, node_attr, mask_downstream_adj, mask_khop_up_adj,
           full_path_edge_attr_adj, outlet_index,
           W_dyn, b_dyn, W_film, b_film, W_pos1, b_pos1, w_pos2, b_pos2,
           W_zr0, b_zr0, W_c0, b_c0, W_zr1, b_zr1, W_c1, b_c1,
           W_out, b_out):
    f32 = jnp.float32

    ea2 = full_path_edge_attr_adj.reshape(_NE, FE)
    md2 = mask_downstream_adj.reshape(_NE, 1)
    mu2 = mask_khop_up_adj.reshape(_NE, 1)
    n_chunks = _NE // _EW_CHUNK
    wspec = lambda shape: pl.BlockSpec(shape, lambda i: (0, 0))
    a_flat = pl.pallas_call(
        _ew_kernel,
        grid=(n_chunks,),
        in_specs=[
            pl.BlockSpec((_EW_CHUNK, FE), lambda i: (i, 0)),
            pl.BlockSpec((_EW_CHUNK, 1), lambda i: (i, 0)),
            pl.BlockSpec((_EW_CHUNK, 1), lambda i: (i, 0)),
            wspec((FE, POS)),
            wspec((1, POS)),
            wspec((POS, 1)),
            wspec((1, 1)),
        ],
        out_specs=pl.BlockSpec((_EW_CHUNK, 1), lambda i: (i, 0)),
        out_shape=jax.ShapeDtypeStruct((_NE, 1), f32),
    )(ea2, md2, mu2, W_pos1, b_pos1.reshape(1, POS), w_pos2,
      b_pos2.reshape(1, 1))
    a = a_flat.reshape(B, N, N)

    at = jnp.swapaxes(a, 1, 2)
    zpad = jnp.zeros((_FPAD - _FIXB, N), f32)
    ab = []
    for b in range(B):
        rows = jnp.stack([a[b, r % N, :] for r, _p in
                          _FIXES[b * _FIXB:(b + 1) * _FIXB]], axis=0)
        ab.append(jnp.concatenate([rows, zpad], axis=0))
    abt = jnp.swapaxes(jnp.concatenate(ab, axis=0), 0, 1)
    pfix = jnp.asarray(_PFIX)

    xtp = jnp.transpose(x, (2, 3, 0, 1)).reshape(T * F, BN)
    na = node_attr.reshape(BN, -1)

    y = pl.pallas_call(
        _gru_kernel,
        out_shape=jax.ShapeDtypeStruct((B, P, O), f32),
        scratch_shapes=[
            pltpu.VMEM((T, BN, H), f32),
            pltpu.VMEM((T, BN, H), f32),
            pltpu.VMEM((BN, H), f32),
            pltpu.VMEM((BN, H), f32),
            pltpu.VMEM((BN, H), f32),
            pltpu.VMEM((BN, H), f32),
            pltpu.VMEM((BN, H), f32),
        ],
    )(at, abt, pfix, xtp, na, outlet_index,
      W_dyn, b_dyn.reshape(1, H), W_film, b_film.reshape(1, 2 * H),
      W_zr0, b_zr0.reshape(1, 2 * H), W_c0, b_c0.reshape(1, H),
      W_zr1, b_zr1.reshape(1, 2 * H), W_c1, b_c1.reshape(1, H),
      W_out, b_out.reshape(1, P))
    return y


# speedup vs baseline: 1.5775x; 1.5775x over previous
import numpy as np

import jax
import jax.numpy as jnp
from jax.experimental import pallas as pl
from jax.experimental.pallas import tpu as pltpu

B = 2
N = 256
T = 24
F = 8
H = 128
FE = 8
POS = 32
O = 32
P = 8
TAILK = 12
BN = B * N

_EW_CHUNK = 8192
_NE = B * N * N

_SHARD = [4320, 4320] + [4080] * 13
_BOUND = np.cumsum(_SHARD)
_FIXES = []
for _b in range(B):
    for _u in _BOUND:
        _p = int(_u % 256)
        if _p != 0:
            _FIXES.append((_b * N + int(_u // 256), _p))
_NFIX = len(_FIXES)
_FIXB = _NFIX // 2
_FPAD = 16
_PFIX = np.full((1, 2 * _FPAD), 256, np.int32)
for _k, (_r, _p) in enumerate(_FIXES):
    _b = _k // _FIXB
    _PFIX[0, _b * _FPAD + (_k % _FIXB)] = _p


def _ew_kernel(ea_ref, md_ref, mu_ref, w1_ref, b1_ref, w2_ref, b2_ref, o_ref):
    h = jnp.dot(ea_ref[...], w1_ref[...], preferred_element_type=jnp.float32)
    h = jnp.maximum(h + b1_ref[...], 0.0)
    s = jnp.dot(h, w2_ref[...], preferred_element_type=jnp.float32) + b2_ref[...]
    ew = jax.nn.sigmoid(s)
    m = jnp.minimum(md_ref[...] + mu_ref[...], 1.0)
    o_ref[...] = m * ew


def _gru_kernel(at_ref, abt_ref, pfix_ref, xtp_ref, na_ref, outlet_ref,
                w_dyn_ref, b_dyn_ref, w_film_ref, b_film_ref,
                wzr0_ref, bzr0_ref, wc0_ref, bc0_ref,
                wzr1_ref, bzr1_ref, wc1_ref, bc1_ref,
                w_out_ref, b_out_ref,
                out_ref,
                s0_ref, s1_ref, hs_ref, xt_ref, ax_ref, ah_ref, acc_ref):
    f32 = jnp.float32

    film = jnp.dot(na_ref[...], w_film_ref[...], preferred_element_type=f32)
    film = film + b_film_ref[...]
    scale = 1.0 + film[:, :H]
    beta = film[:, H:]
    w_dyn = w_dyn_ref[...]
    b_dyn = b_dyn_ref[...]
    for t in range(T):
        xt = xtp_ref[t * F:(t + 1) * F, :]
        ht = jax.lax.dot_general(xt, w_dyn, (((0,), (0,)), ((), ())),
                                 preferred_element_type=f32)
        s0_ref[t] = jnp.maximum((ht + b_dyn) * scale + beta, 0.0)

    pfix = pfix_ref[...]

    def gprop(src_ref, dst_ref):
        for b in range(B):
            pfb = pfix[:, b * _FPAD:(b + 1) * _FPAD]

            def gbody(g, carry, b=b, pfb=pfb):
                accT, floT, fhiT = carry
                blk = at_ref[b, pl.ds(g * 8, 8), :]
                fblk = abt_ref[pl.ds(g * 8, 8),
                               b * _FPAD:(b + 1) * _FPAD]
                hb = src_ref[pl.ds(b * N + g * 8, 8), :]
                hT = jnp.swapaxes(hb, 0, 1)
                sel = (g * 8 < pfb).astype(f32)
                selc = 1.0 - sel
                for s in range(8):
                    row = blk[s:s + 1, :]
                    hc = hT[:, s:s + 1]
                    accT = accT + hc * row
                    mfx = hc * fblk[s:s + 1, :]
                    floT = floT + mfx * sel
                    fhiT = fhiT + mfx * selc
                return accT, floT, fhiT

            zA = jnp.zeros((H, N), f32)
            zF = jnp.zeros((H, _FPAD), f32)
            accT, floT, fhiT = jax.lax.fori_loop(0, N // 8, gbody,
                                                 (zA, zF, zF))
            dst_ref[b * N:(b + 1) * N, :] = jnp.swapaxes(accT, 0, 1)
            fix = jnp.swapaxes(floT + fhiT, 0, 1)
            for k in range(_FIXB):
                r = _FIXES[b * _FIXB + k][0]
                dst_ref[r:r + 1, :] = fix[k:k + 1, :]

    acc_ref[...] = jnp.zeros((BN, H), f32)
    for layer in range(2):
        wzr = (wzr0_ref if layer == 0 else wzr1_ref)[...]
        bzr = (bzr0_ref if layer == 0 else bzr1_ref)[...]
        wc = (wc0_ref if layer == 0 else wc1_ref)[...]
        bc = (bc0_ref if layer == 0 else bc1_ref)[...]
        s_in = s0_ref if layer == 0 else s1_ref
        hs_ref[...] = jnp.zeros((BN, H), f32)

        def step(t, _, s_in=s_in, wzr=wzr, bzr=bzr, wc=wc, bc=bc, layer=layer):
            xt_ref[...] = s_in[t]
            gprop(xt_ref, ax_ref)
            gprop(hs_ref, ah_ref)
            xt = xt_ref[...]
            h = hs_ref[...]
            agg_x = ax_ref[...]
            agg_h = ah_ref[...]
            m = jnp.concatenate([xt, agg_x, h, agg_h], axis=1)
            zr = jax.nn.sigmoid(jnp.dot(m, wzr, preferred_element_type=f32)
                                + bzr)
            z = zr[:, :H]
            r = zr[:, H:]
            mc = jnp.concatenate([xt, agg_x, r * h, r * agg_h], axis=1)
            c = jnp.tanh(jnp.dot(mc, wc, preferred_element_type=f32) + bc)
            hnew = z * h + (1.0 - z) * c
            hs_ref[...] = hnew
            if layer == 0:
                s1_ref[t] = hnew
            else:
                acc_ref[...] = acc_ref[...] + jnp.where(t >= T - TAILK,
                                                        hnew, 0.0)
            return 0

        jax.lax.fori_loop(0, T, step, 0)

    tail = acc_ref[...] * (1.0 / TAILK)
    npred = jnp.dot(tail, w_out_ref[...], preferred_element_type=f32)
    npred = npred + b_out_ref[...]
    iota = jax.lax.broadcasted_iota(jnp.int32, (N, O), 0)
    for b in range(B):
        idx = outlet_ref[b:b + 1, :]
        onehot = (iota == idx).astype(f32)
        nb = npred[b * N:(b + 1) * N, :]
        yb = jax.lax.dot_general(nb, onehot, (((0,), (0,)), ((), ())),
                                 precision=jax.lax.Precision.HIGHEST,
                                 preferred_element_type=f32)
        out_ref[b] = yb


@jax.jit
def kernel(x, node_attr, mask_downstream_adj, mask_khop_up_adj,
           full_path_edge_attr_adj, outlet_index,
           W_dyn, b_dyn, W_film, b_film, W_pos1, b_pos1, w_pos2, b_pos2,
           W_zr0, b_zr0, W_c0, b_c0, W_zr1, b_zr1, W_c1, b_c1,
           W_out, b_out):
    f32 = jnp.float32

    ea2 = full_path_edge_attr_adj.reshape(_NE, FE)
    md2 = mask_downstream_adj.reshape(_NE, 1)
    mu2 = mask_khop_up_adj.reshape(_NE, 1)
    n_chunks = _NE // _EW_CHUNK
    wspec = lambda shape: pl.BlockSpec(shape, lambda i: (0, 0))
    a_flat = pl.pallas_call(
        _ew_kernel,
        grid=(n_chunks,),
        in_specs=[
            pl.BlockSpec((_EW_CHUNK, FE), lambda i: (i, 0)),
            pl.BlockSpec((_EW_CHUNK, 1), lambda i: (i, 0)),
            pl.BlockSpec((_EW_CHUNK, 1), lambda i: (i, 0)),
            wspec((FE, POS)),
            wspec((1, POS)),
            wspec((POS, 1)),
            wspec((1, 1)),
        ],
        out_specs=pl.BlockSpec((_EW_CHUNK, 1), lambda i: (i, 0)),
        out_shape=jax.ShapeDtypeStruct((_NE, 1), f32),
    )(ea2, md2, mu2, W_pos1, b_pos1.reshape(1, POS), w_pos2,
      b_pos2.reshape(1, 1))
    a = a_flat.reshape(B, N, N)

    at = jnp.swapaxes(a, 1, 2)
    zpad = jnp.zeros((_FPAD - _FIXB, N), f32)
    ab = []
    for b in range(B):
        rows = jnp.stack([a[b, r % N, :] for r, _p in
                          _FIXES[b * _FIXB:(b + 1) * _FIXB]], axis=0)
        ab.append(jnp.concatenate([rows, zpad], axis=0))
    abt = jnp.swapaxes(jnp.concatenate(ab, axis=0), 0, 1)
    pfix = jnp.asarray(_PFIX)

    xtp = jnp.transpose(x, (2, 3, 0, 1)).reshape(T * F, BN)
    na = node_attr.reshape(BN, -1)

    y = pl.pallas_call(
        _gru_kernel,
        out_shape=jax.ShapeDtypeStruct((B, P, O), f32),
        scratch_shapes=[
            pltpu.VMEM((T, BN, H), f32),
            pltpu.VMEM((T, BN, H), f32),
            pltpu.VMEM((BN, H), f32),
            pltpu.VMEM((BN, H), f32),
            pltpu.VMEM((BN, H), f32),
            pltpu.VMEM((BN, H), f32),
            pltpu.VMEM((BN, H), f32),
        ],
    )(at, abt, pfix, xtp, na, outlet_index,
      W_dyn, b_dyn.reshape(1, H), W_film, b_film.reshape(1, 2 * H),
      W_zr0, b_zr0.reshape(1, 2 * H), W_c0, b_c0.reshape(1, H),
      W_zr1, b_zr1.reshape(1, 2 * H), W_c1, b_c1.reshape(1, H),
      W_out, b_out.reshape(1, P))
    return y


# speedup vs baseline: 1.7362x; 1.1006x over previous
import numpy as np

import jax
import jax.numpy as jnp
from jax.experimental import pallas as pl
from jax.experimental.pallas import tpu as pltpu

B = 2
N = 256
T = 24
F = 8
H = 128
FE = 8
POS = 32
O = 32
P = 8
TAILK = 12
BN = B * N

_EW_CHUNK = 8192
_NE = B * N * N

_SHARD = [4320, 4320] + [4080] * 13
_BOUND = np.cumsum(_SHARD)
_FIXES = []
for _b in range(B):
    for _u in _BOUND:
        _p = int(_u % 256)
        if _p != 0:
            _FIXES.append((_b * N + int(_u // 256), _p))
_NFIX = len(_FIXES)
_FIXB = _NFIX // 2
_FPAD = 16
_PFIX = np.full((1, 2 * _FPAD), 256, np.int32)
for _k, (_r, _p) in enumerate(_FIXES):
    _b = _k // _FIXB
    _PFIX[0, _b * _FPAD + (_k % _FIXB)] = _p


def _ew_kernel(ea_ref, md_ref, mu_ref, w1_ref, b1_ref, w2_ref, b2_ref, o_ref):
    h = jnp.dot(ea_ref[...], w1_ref[...], preferred_element_type=jnp.float32)
    h = jnp.maximum(h + b1_ref[...], 0.0)
    s = jnp.dot(h, w2_ref[...], preferred_element_type=jnp.float32) + b2_ref[...]
    ew = jax.nn.sigmoid(s)
    m = jnp.minimum(md_ref[...] + mu_ref[...], 1.0)
    o_ref[...] = m * ew


def _gru_kernel(at_ref, abt_ref, pfix_ref, xtp_ref, na_ref, outlet_ref,
                w_dyn_ref, b_dyn_ref, w_film_ref, b_film_ref,
                wzr0_ref, bzr0_ref, wc0_ref, bc0_ref,
                wzr1_ref, bzr1_ref, wc1_ref, bc1_ref,
                w_out_ref, b_out_ref,
                out_ref,
                s0_ref, s1_ref, hs_ref, xt_ref, ax_ref, ah_ref, acc_ref):
    f32 = jnp.float32

    film = jnp.dot(na_ref[...], w_film_ref[...], preferred_element_type=f32)
    film = film + b_film_ref[...]
    scale = 1.0 + film[:, :H]
    beta = film[:, H:]
    w_dyn = w_dyn_ref[...]
    b_dyn = b_dyn_ref[...]
    for t in range(T):
        xt = xtp_ref[t * F:(t + 1) * F, :]
        ht = jax.lax.dot_general(xt, w_dyn, (((0,), (0,)), ((), ())),
                                 preferred_element_type=f32)
        s0_ref[t] = jnp.maximum((ht + b_dyn) * scale + beta, 0.0)

    pfix = pfix_ref[...]

    def gprop2(x_ref, h_ref, ax_dst, ah_dst):
        for b in range(B):
            pfb = pfix[:, b * _FPAD:(b + 1) * _FPAD]

            def gbody(g, carry, b=b, pfb=pfb):
                accX, accH, floX, fhiX, floH, fhiH = carry
                blk = at_ref[b, pl.ds(g * 8, 8), :]
                fblk = abt_ref[pl.ds(g * 8, 8),
                               b * _FPAD:(b + 1) * _FPAD]
                xb = x_ref[pl.ds(b * N + g * 8, 8), :]
                hb = h_ref[pl.ds(b * N + g * 8, 8), :]
                xT = jnp.swapaxes(xb, 0, 1)
                hT = jnp.swapaxes(hb, 0, 1)
                sel = (g * 8 < pfb).astype(f32)
                selc = 1.0 - sel
                for s in range(8):
                    row = blk[s:s + 1, :]
                    frow = fblk[s:s + 1, :]
                    xc = xT[:, s:s + 1]
                    hc = hT[:, s:s + 1]
                    accX = accX + xc * row
                    accH = accH + hc * row
                    mfxX = xc * frow
                    mfxH = hc * frow
                    floX = floX + mfxX * sel
                    fhiX = fhiX + mfxX * selc
                    floH = floH + mfxH * sel
                    fhiH = fhiH + mfxH * selc
                return accX, accH, floX, fhiX, floH, fhiH

            zA = jnp.zeros((H, N), f32)
            zF = jnp.zeros((H, _FPAD), f32)
            accX, accH, floX, fhiX, floH, fhiH = jax.lax.fori_loop(
                0, N // 8, gbody, (zA, zA, zF, zF, zF, zF))
            ax_dst[b * N:(b + 1) * N, :] = jnp.swapaxes(accX, 0, 1)
            ah_dst[b * N:(b + 1) * N, :] = jnp.swapaxes(accH, 0, 1)
            fixX = jnp.swapaxes(floX + fhiX, 0, 1)
            fixH = jnp.swapaxes(floH + fhiH, 0, 1)
            for k in range(_FIXB):
                r = _FIXES[b * _FIXB + k][0]
                ax_dst[r:r + 1, :] = fixX[k:k + 1, :]
                ah_dst[r:r + 1, :] = fixH[k:k + 1, :]

    acc_ref[...] = jnp.zeros((BN, H), f32)
    for layer in range(2):
        wzr = (wzr0_ref if layer == 0 else wzr1_ref)[...]
        bzr = (bzr0_ref if layer == 0 else bzr1_ref)[...]
        wc = (wc0_ref if layer == 0 else wc1_ref)[...]
        bc = (bc0_ref if layer == 0 else bc1_ref)[...]
        s_in = s0_ref if layer == 0 else s1_ref
        hs_ref[...] = jnp.zeros((BN, H), f32)

        def step(t, _, s_in=s_in, wzr=wzr, bzr=bzr, wc=wc, bc=bc, layer=layer):
            xt_ref[...] = s_in[t]
            gprop2(xt_ref, hs_ref, ax_ref, ah_ref)
            xt = xt_ref[...]
            h = hs_ref[...]
            agg_x = ax_ref[...]
            agg_h = ah_ref[...]
            m = jnp.concatenate([xt, agg_x, h, agg_h], axis=1)
            zr = jax.nn.sigmoid(jnp.dot(m, wzr, preferred_element_type=f32)
                                + bzr)
            z = zr[:, :H]
            r = zr[:, H:]
            mc = jnp.concatenate([xt, agg_x, r * h, r * agg_h], axis=1)
            c = jnp.tanh(jnp.dot(mc, wc, preferred_element_type=f32) + bc)
            hnew = z * h + (1.0 - z) * c
            hs_ref[...] = hnew
            if layer == 0:
                s1_ref[t] = hnew
            else:
                acc_ref[...] = acc_ref[...] + jnp.where(t >= T - TAILK,
                                                        hnew, 0.0)
            return 0

        jax.lax.fori_loop(0, T, step, 0)

    tail = acc_ref[...] * (1.0 / TAILK)
    npred = jnp.dot(tail, w_out_ref[...], preferred_element_type=f32)
    npred = npred + b_out_ref[...]
    iota = jax.lax.broadcasted_iota(jnp.int32, (N, O), 0)
    for b in range(B):
        idx = outlet_ref[b:b + 1, :]
        onehot = (iota == idx).astype(f32)
        nb = npred[b * N:(b + 1) * N, :]
        yb = jax.lax.dot_general(nb, onehot, (((0,), (0,)), ((), ())),
                                 precision=jax.lax.Precision.HIGHEST,
                                 preferred_element_type=f32)
        out_ref[b] = yb


@jax.jit
def kernel(x, node_attr, mask_downstream_adj, mask_khop_up_adj,
           full_path_edge_attr_adj, outlet_index,
           W_dyn, b_dyn, W_film, b_film, W_pos1, b_pos1, w_pos2, b_pos2,
           W_zr0, b_zr0, W_c0, b_c0, W_zr1, b_zr1, W_c1, b_c1,
           W_out, b_out):
    f32 = jnp.float32

    ea2 = full_path_edge_attr_adj.reshape(_NE, FE)
    md2 = mask_downstream_adj.reshape(_NE, 1)
    mu2 = mask_khop_up_adj.reshape(_NE, 1)
    n_chunks = _NE // _EW_CHUNK
    wspec = lambda shape: pl.BlockSpec(shape, lambda i: (0, 0))
    a_flat = pl.pallas_call(
        _ew_kernel,
        grid=(n_chunks,),
        in_specs=[
            pl.BlockSpec((_EW_CHUNK, FE), lambda i: (i, 0)),
            pl.BlockSpec((_EW_CHUNK, 1), lambda i: (i, 0)),
            pl.BlockSpec((_EW_CHUNK, 1), lambda i: (i, 0)),
            wspec((FE, POS)),
            wspec((1, POS)),
            wspec((POS, 1)),
            wspec((1, 1)),
        ],
        out_specs=pl.BlockSpec((_EW_CHUNK, 1), lambda i: (i, 0)),
        out_shape=jax.ShapeDtypeStruct((_NE, 1), f32),
    )(ea2, md2, mu2, W_pos1, b_pos1.reshape(1, POS), w_pos2,
      b_pos2.reshape(1, 1))
    a = a_flat.reshape(B, N, N)

    at = jnp.swapaxes(a, 1, 2)
    zpad = jnp.zeros((_FPAD - _FIXB, N), f32)
    ab = []
    for b in range(B):
        rows = jnp.stack([a[b, r % N, :] for r, _p in
                          _FIXES[b * _FIXB:(b + 1) * _FIXB]], axis=0)
        ab.append(jnp.concatenate([rows, zpad], axis=0))
    abt = jnp.swapaxes(jnp.concatenate(ab, axis=0), 0, 1)
    pfix = jnp.asarray(_PFIX)

    xtp = jnp.transpose(x, (2, 3, 0, 1)).reshape(T * F, BN)
    na = node_attr.reshape(BN, -1)

    y = pl.pallas_call(
        _gru_kernel,
        out_shape=jax.ShapeDtypeStruct((B, P, O), f32),
        scratch_shapes=[
            pltpu.VMEM((T, BN, H), f32),
            pltpu.VMEM((T, BN, H), f32),
            pltpu.VMEM((BN, H), f32),
            pltpu.VMEM((BN, H), f32),
            pltpu.VMEM((BN, H), f32),
            pltpu.VMEM((BN, H), f32),
            pltpu.VMEM((BN, H), f32),
        ],
    )(at, abt, pfix, xtp, na, outlet_index,
      W_dyn, b_dyn.reshape(1, H), W_film, b_film.reshape(1, 2 * H),
      W_zr0, b_zr0.reshape(1, 2 * H), W_c0, b_c0.reshape(1, H),
      W_zr1, b_zr1.reshape(1, 2 * H), W_c1, b_c1.reshape(1, H),
      W_out, b_out.reshape(1, P))
    return y


# speedup vs baseline: 1.7581x; 1.0126x over previous
import numpy as np

import jax
import jax.numpy as jnp
from jax.experimental import pallas as pl
from jax.experimental.pallas import tpu as pltpu

B = 2
N = 256
T = 24
F = 8
H = 128
FE = 8
POS = 32
O = 32
P = 8
TAILK = 12
BN = B * N

_EW_CHUNK = 8192
_NE = B * N * N

_SHARD = [4320, 4320] + [4080] * 13
_BOUND = np.cumsum(_SHARD)
_FIXES = []
for _b in range(B):
    for _u in _BOUND:
        _p = int(_u % 256)
        if _p != 0:
            _FIXES.append((_b * N + int(_u // 256), _p))
_NFIX = len(_FIXES)
_FIXB = _NFIX // 2
_FPAD = 16
_PFIX = np.full((1, 2 * _FPAD), 256, np.int32)
for _k, (_r, _p) in enumerate(_FIXES):
    _b = _k // _FIXB
    _PFIX[0, _b * _FPAD + (_k % _FIXB)] = _p


def _ew_kernel(ea_ref, md_ref, mu_ref, w1_ref, b1_ref, w2_ref, b2_ref, o_ref):
    h = jnp.dot(ea_ref[...], w1_ref[...], preferred_element_type=jnp.float32)
    h = jnp.maximum(h + b1_ref[...], 0.0)
    s = jnp.dot(h, w2_ref[...], preferred_element_type=jnp.float32) + b2_ref[...]
    ew = jax.nn.sigmoid(s)
    m = jnp.minimum(md_ref[...] + mu_ref[...], 1.0)
    o_ref[...] = m * ew


def _gru_kernel(at_ref, abt_ref, pfix_ref, xtp_ref, na_ref, outlet_ref,
                w_dyn_ref, b_dyn_ref, w_film_ref, b_film_ref,
                wzr0_ref, bzr0_ref, wc0_ref, bc0_ref,
                wzr1_ref, bzr1_ref, wc1_ref, bc1_ref,
                w_out_ref, b_out_ref,
                out_ref,
                s0_ref, s1_ref, hs_ref, xt_ref, ax_ref, ah_ref, acc_ref):
    f32 = jnp.float32

    film = jnp.dot(na_ref[...], w_film_ref[...], preferred_element_type=f32)
    film = film + b_film_ref[...]
    scale = 1.0 + film[:, :H]
    beta = film[:, H:]
    w_dyn = w_dyn_ref[...]
    b_dyn = b_dyn_ref[...]
    for t in range(T):
        xt = xtp_ref[t * F:(t + 1) * F, :]
        ht = jax.lax.dot_general(xt, w_dyn, (((0,), (0,)), ((), ())),
                                 preferred_element_type=f32)
        s0_ref[t] = jnp.maximum((ht + b_dyn) * scale + beta, 0.0)

    pfix = pfix_ref[...]

    def gprop2(x_ref, h_ref, ax_dst, ah_dst):
        for b in range(B):
            pfb = pfix[:, b * _FPAD:(b + 1) * _FPAD]

            def gbody(g, carry, b=b, pfb=pfb):
                accX, accH, floX, fhiX, floH, fhiH = carry
                blk = at_ref[b, pl.ds(g * 8, 8), :]
                fblk = abt_ref[pl.ds(g * 8, 8),
                               b * _FPAD:(b + 1) * _FPAD]
                xb = x_ref[pl.ds(b * N + g * 8, 8), :]
                hb = h_ref[pl.ds(b * N + g * 8, 8), :]
                xT = jnp.swapaxes(xb, 0, 1)
                hT = jnp.swapaxes(hb, 0, 1)
                sel = (g * 8 < pfb).astype(f32)
                flo_blk = fblk * sel
                fhi_blk = fblk * (1.0 - sel)
                for s in range(8):
                    row = blk[s:s + 1, :]
                    xc = xT[:, s:s + 1]
                    hc = hT[:, s:s + 1]
                    accX = accX + xc * row
                    accH = accH + hc * row
                    floX = floX + xc * flo_blk[s:s + 1, :]
                    fhiX = fhiX + xc * fhi_blk[s:s + 1, :]
                    floH = floH + hc * flo_blk[s:s + 1, :]
                    fhiH = fhiH + hc * fhi_blk[s:s + 1, :]
                return accX, accH, floX, fhiX, floH, fhiH

            zA = jnp.zeros((H, N), f32)
            zF = jnp.zeros((H, _FPAD), f32)
            accX, accH, floX, fhiX, floH, fhiH = jax.lax.fori_loop(
                0, N // 8, gbody, (zA, zA, zF, zF, zF, zF))
            ax_dst[b * N:(b + 1) * N, :] = jnp.swapaxes(accX, 0, 1)
            ah_dst[b * N:(b + 1) * N, :] = jnp.swapaxes(accH, 0, 1)
            fixX = jnp.swapaxes(floX + fhiX, 0, 1)
            fixH = jnp.swapaxes(floH + fhiH, 0, 1)
            for k in range(_FIXB):
                r = _FIXES[b * _FIXB + k][0]
                ax_dst[r:r + 1, :] = fixX[k:k + 1, :]
                ah_dst[r:r + 1, :] = fixH[k:k + 1, :]

    acc_ref[...] = jnp.zeros((BN, H), f32)
    for layer in range(2):
        wzr = (wzr0_ref if layer == 0 else wzr1_ref)[...]
        bzr = (bzr0_ref if layer == 0 else bzr1_ref)[...]
        wc = (wc0_ref if layer == 0 else wc1_ref)[...]
        bc = (bc0_ref if layer == 0 else bc1_ref)[...]
        s_in = s0_ref if layer == 0 else s1_ref
        hs_ref[...] = jnp.zeros((BN, H), f32)

        def step(t, _, s_in=s_in, wzr=wzr, bzr=bzr, wc=wc, bc=bc, layer=layer):
            xt_ref[...] = s_in[t]
            gprop2(xt_ref, hs_ref, ax_ref, ah_ref)
            xt = xt_ref[...]
            h = hs_ref[...]
            agg_x = ax_ref[...]
            agg_h = ah_ref[...]
            m = jnp.concatenate([xt, agg_x, h, agg_h], axis=1)
            zr = jax.nn.sigmoid(jnp.dot(m, wzr, preferred_element_type=f32)
                                + bzr)
            z = zr[:, :H]
            r = zr[:, H:]
            mc = jnp.concatenate([xt, agg_x, r * h, r * agg_h], axis=1)
            c = jnp.tanh(jnp.dot(mc, wc, preferred_element_type=f32) + bc)
            hnew = z * h + (1.0 - z) * c
            hs_ref[...] = hnew
            if layer == 0:
                s1_ref[t] = hnew
            else:
                acc_ref[...] = acc_ref[...] + jnp.where(t >= T - TAILK,
                                                        hnew, 0.0)
            return 0

        jax.lax.fori_loop(0, T, step, 0)

    tail = acc_ref[...] * (1.0 / TAILK)
    npred = jnp.dot(tail, w_out_ref[...], preferred_element_type=f32)
    npred = npred + b_out_ref[...]
    iota = jax.lax.broadcasted_iota(jnp.int32, (N, O), 0)
    for b in range(B):
        idx = outlet_ref[b:b + 1, :]
        onehot = (iota == idx).astype(f32)
        nb = npred[b * N:(b + 1) * N, :]
        yb = jax.lax.dot_general(nb, onehot, (((0,), (0,)), ((), ())),
                                 precision=jax.lax.Precision.HIGHEST,
                                 preferred_element_type=f32)
        out_ref[b] = yb


@jax.jit
def kernel(x, node_attr, mask_downstream_adj, mask_khop_up_adj,
           full_path_edge_attr_adj, outlet_index,
           W_dyn, b_dyn, W_film, b_film, W_pos1, b_pos1, w_pos2, b_pos2,
           W_zr0, b_zr0, W_c0, b_c0, W_zr1, b_zr1, W_c1, b_c1,
           W_out, b_out):
    f32 = jnp.float32

    ea2 = full_path_edge_attr_adj.reshape(_NE, FE)
    md2 = mask_downstream_adj.reshape(_NE, 1)
    mu2 = mask_khop_up_adj.reshape(_NE, 1)
    n_chunks = _NE // _EW_CHUNK
    wspec = lambda shape: pl.BlockSpec(shape, lambda i: (0, 0))
    a_flat = pl.pallas_call(
        _ew_kernel,
        grid=(n_chunks,),
        in_specs=[
            pl.BlockSpec((_EW_CHUNK, FE), lambda i: (i, 0)),
            pl.BlockSpec((_EW_CHUNK, 1), lambda i: (i, 0)),
            pl.BlockSpec((_EW_CHUNK, 1), lambda i: (i, 0)),
            wspec((FE, POS)),
            wspec((1, POS)),
            wspec((POS, 1)),
            wspec((1, 1)),
        ],
        out_specs=pl.BlockSpec((_EW_CHUNK, 1), lambda i: (i, 0)),
        out_shape=jax.ShapeDtypeStruct((_NE, 1), f32),
    )(ea2, md2, mu2, W_pos1, b_pos1.reshape(1, POS), w_pos2,
      b_pos2.reshape(1, 1))
    a = a_flat.reshape(B, N, N)

    at = jnp.swapaxes(a, 1, 2)
    zpad = jnp.zeros((_FPAD - _FIXB, N), f32)
    ab = []
    for b in range(B):
        rows = jnp.stack([a[b, r % N, :] for r, _p in
                          _FIXES[b * _FIXB:(b + 1) * _FIXB]], axis=0)
        ab.append(jnp.concatenate([rows, zpad], axis=0))
    abt = jnp.swapaxes(jnp.concatenate(ab, axis=0), 0, 1)
    pfix = jnp.asarray(_PFIX)

    xtp = jnp.transpose(x, (2, 3, 0, 1)).reshape(T * F, BN)
    na = node_attr.reshape(BN, -1)

    y = pl.pallas_call(
        _gru_kernel,
        out_shape=jax.ShapeDtypeStruct((B, P, O), f32),
        scratch_shapes=[
            pltpu.VMEM((T, BN, H), f32),
            pltpu.VMEM((T, BN, H), f32),
            pltpu.VMEM((BN, H), f32),
            pltpu.VMEM((BN, H), f32),
            pltpu.VMEM((BN, H), f32),
            pltpu.VMEM((BN, H), f32),
            pltpu.VMEM((BN, H), f32),
        ],
    )(at, abt, pfix, xtp, na, outlet_index,
      W_dyn, b_dyn.reshape(1, H), W_film, b_film.reshape(1, 2 * H),
      W_zr0, b_zr0.reshape(1, 2 * H), W_c0, b_c0.reshape(1, H),
      W_zr1, b_zr1.reshape(1, 2 * H), W_c1, b_c1.reshape(1, H),
      W_out, b_out.reshape(1, P))
    return y
